# baseline (speedup 1.0000x reference)
import functools

import jax
import jax.numpy as jnp
from jax import lax
from jax.experimental import pallas as pl
from jax.experimental.pallas import tpu as pltpu
from jax.experimental.pallas import tpu_sc as plsc

_VOCAB = 1000000
_D = 32
_NB = 4096
_H = 50
_NW = 32
_LN = 128
_TCOLS = 7808


def _iota16():
    return lax.broadcasted_iota(jnp.int32, (16,), 0)


def _make_repack():
    mesh = plsc.VectorSubcoreMesh(core_axis_name="c", subcore_axis_name="s")

    @functools.partial(
        pl.kernel,
        mesh=mesh,
        out_type=jax.ShapeDtypeStruct((_VOCAB * _D,), jnp.float32),
        scratch_types=[
            pltpu.VMEM((64, 2 * _LN), jnp.float32),
            pltpu.VMEM((2 * 2 * _LN * _D,), jnp.float32),
            [pltpu.SemaphoreType.DMA] * 2,
            [pltpu.SemaphoreType.DMA] * 2,
        ],
        compiler_params=pltpu.CompilerParams(needs_layout_passes=False),
    )
    def k(table_t, tail_in, out_rm, sbuf, lbuf, gsems, wsems):
        wid = lax.axis_index("s") * 2 + lax.axis_index("c")
        iota = _iota16()
        _PW = 2 * _LN
        _PB = _PW * _D
        diags = [(iota + j) & 15 for j in range(16)]
        rows = {(b, ci): iota + (b * _D + ci * 16)
                for b in range(2) for ci in range(2)}
        cbase = {ci: iota + ci * 16 for ci in range(2)}

        def fetch(p, b):
            pltpu.async_copy(
                table_t.at[:, pl.ds(p * _PW, _PW)],
                sbuf.at[pl.ds(b * _D, _D), :],
                gsems[b],
            )

        def fetch_wait(p, b):
            pltpu.make_async_copy(
                table_t.at[:, pl.ds(p * _PW, _PW)],
                sbuf.at[pl.ds(b * _D, _D), :],
                gsems[b],
            ).wait()

        def transpose(b, nli):
            def body(li, carry):
                l0 = li * 16
                for ci in range(2):
                    row = rows[(b, ci)]
                    cb = cbase[ci]
                    for j in range(16):
                        col = diags[j] + l0
                        v = plsc.load_gather(sbuf, [row, col])
                        plsc.store_scatter(
                            lbuf,
                            [(col << 5) + cb + b * _PB],
                            v,
                        )
                return carry

            lax.fori_loop(0, nli, body, 0)

        def write(p, b):
            return (
                lbuf.at[pl.ds(b * _PB, _PB)],
                out_rm.at[pl.ds(p * _PB, _PB)],
                wsems[b],
            )

        fetch(wid, 0)
        fetch(wid + _NW, 1)

        def outer(u, carry):
            for b in range(2):
                t = u * 2 + b
                p = wid + t * _NW
                fetch_wait(p, b)

                @pl.when(u >= 1)
                def _():
                    pltpu.make_async_copy(*write(p - 2 * _NW, b)).wait()

                transpose(b, _PW // 16)
                pltpu.async_copy(*write(p, b))

                @pl.when(t + 2 < 122)
                def _():
                    fetch(p + 2 * _NW, b)

            return carry

        lax.fori_loop(0, 61, outer, 0)
        for b in range(2):
            pltpu.make_async_copy(*write(wid + (120 + b) * _NW, b)).wait()

        @pl.when(wid < 2)
        def _():
            p = 3904 + wid
            fetch(p, 0)
            fetch_wait(p, 0)
            transpose(0, _PW // 16)
            pltpu.async_copy(*write(p, 0))
            pltpu.make_async_copy(*write(p, 0)).wait()

        @pl.when(wid == 4)
        def _():
            pltpu.sync_copy(tail_in, sbuf.at[pl.ds(0, 16), pl.ds(0, _LN)])

            def body(r, carry):
                for seg in range(8):
                    v = plsc.load_gather(sbuf, [lax.broadcast(r, (16,)),
                                                iota + seg * 16])
                    plsc.store_scatter(lbuf, [iota + seg * 16 + r * _LN], v)
                return carry

            lax.fori_loop(0, 16, body, 0)
            pltpu.async_copy(
                lbuf.at[pl.ds(0, 64 * _D)],
                out_rm.at[pl.ds((_VOCAB - 64) * _D, 64 * _D)],
                wsems[0],
            )
            pltpu.make_async_copy(
                lbuf.at[pl.ds(0, 64 * _D)],
                out_rm.at[pl.ds((_VOCAB - 64) * _D, 64 * _D)],
                wsems[0],
            ).wait()

    return k


def _make_gather():
    mesh = plsc.VectorSubcoreMesh(core_axis_name="c", subcore_axis_name="s")

    @functools.partial(
        pl.kernel,
        mesh=mesh,
        out_type=jax.ShapeDtypeStruct((_H, _D // 8, _NB // _LN, 8, _LN),
                                      jnp.float32),
        scratch_types=[
            pltpu.VMEM((2 * _LN,), jnp.int32),
            pltpu.VMEM((2 * _LN, _D), jnp.float32),
            pltpu.VMEM((8, 8, _LN), jnp.float32),
            [pltpu.SemaphoreType.DMA] * 2,
            [pltpu.SemaphoreType.DMA] * 2,
        ],
        compiler_params=pltpu.CompilerParams(
            use_tc_tiling_on_sc=False, needs_layout_passes=False
        ),
    )
    def k(table_rm, idx_t, out5, idx_v, gbuf, tbuf, gsems, wsems):
        wid = lax.axis_index("s") * 2 + lax.axis_index("c")
        iota = _iota16()
        diags = [(iota + j) & 15 for j in range(16)]
        t_c8 = iota % 8
        t_ct = {ci: iota // 8 + ci * 2 for ci in range(2)}
        cvec = {ci: iota + ci * 16 for ci in range(2)}

        def stage(h, b):
            pltpu.sync_copy(
                idx_t.at[h, pl.ds(wid * _LN, _LN)],
                idx_v.at[pl.ds(b * _LN, _LN)],
            )
            pltpu.async_copy(
                table_rm.at[idx_v.at[pl.ds(b * _LN, _LN)]],
                gbuf.at[pl.ds(b * _LN, _LN), :],
                gsems[b],
            )

        def gather_wait(b):
            pltpu.make_async_copy(
                table_rm.at[idx_v.at[pl.ds(b * _LN, _LN)]],
                gbuf.at[pl.ds(b * _LN, _LN), :],
                gsems[b],
            ).wait()

        def wcopy(h, b):
            return (
                tbuf.at[pl.ds(b * 4, 4)],
                out5.at[h, :, wid],
                wsems[b],
            )

        stage(0, 0)

        def outer(u, carry):
            for b in range(2):
                h = u * 2 + b
                nb = 1 - b
                gather_wait(b)

                @pl.when(h + 1 < _H)
                def _():
                    stage(h + 1, nb)

                @pl.when(h >= 2)
                def _():
                    pltpu.make_async_copy(*wcopy(h - 2, b)).wait()

                def trow(lp, carry2):
                    l0 = lp * 16
                    for ci in range(2):
                        for j in range(16):
                            lvec = diags[j] + l0
                            v = plsc.load_gather(
                                gbuf, [lvec + b * _LN, cvec[ci]]
                            )
                            plsc.store_scatter(
                                tbuf,
                                [t_ct[ci] + b * 4, t_c8, lvec],
                                v,
                            )
                    return carry2

                lax.fori_loop(0, _LN // 16, trow, 0)
                pltpu.async_copy(*wcopy(h, b))
            return carry

        lax.fori_loop(0, _H // 2, outer, 0)
        for h in (_H - 2, _H - 1):
            pltpu.make_async_copy(*wcopy(h, h % 2)).wait()

    return k


_repack = _make_repack()
_gather = _make_gather()


def kernel(inputs, embeddings):
    tail128 = embeddings[_VOCAB - 64:, :].reshape(16, _LN)
    table_rm = _repack(embeddings.T, tail128)
    out5 = _gather(
        table_rm.reshape(_VOCAB, _D), inputs.T.astype(jnp.int32)
    )
    return out5.transpose(2, 4, 0, 1, 3).reshape(_NB, _H, _D)


# speedup vs baseline: 1.0098x; 1.0098x over previous
import functools

import jax
import jax.numpy as jnp
from jax import lax
from jax.experimental import pallas as pl
from jax.experimental.pallas import tpu as pltpu
from jax.experimental.pallas import tpu_sc as plsc

_VOCAB = 1000000
_D = 32
_NB = 4096
_H = 50
_NW = 32
_LN = 128
_TCOLS = 7808


def _iota16():
    return lax.broadcasted_iota(jnp.int32, (16,), 0)


def _make_repack():
    mesh = plsc.VectorSubcoreMesh(core_axis_name="c", subcore_axis_name="s")

    @functools.partial(
        pl.kernel,
        mesh=mesh,
        out_type=jax.ShapeDtypeStruct((_VOCAB * _D,), jnp.float32),
        scratch_types=[
            pltpu.VMEM((64, _LN), jnp.float32),
            pltpu.VMEM((2 * _LN * _D,), jnp.float32),
            [pltpu.SemaphoreType.DMA] * 2,
            [pltpu.SemaphoreType.DMA] * 2,
        ],
        compiler_params=pltpu.CompilerParams(needs_layout_passes=False),
    )
    def k(table_t, tail_in, out_rm, sbuf, lbuf, gsems, wsems):
        wid = lax.axis_index("s") * 2 + lax.axis_index("c")
        iota = _iota16()
        _PW = _LN
        _PB = _PW * _D
        diags = [(iota + j) & 15 for j in range(16)]
        rows = {(b, ci): iota + (b * _D + ci * 16)
                for b in range(2) for ci in range(2)}
        cbase = {ci: iota + ci * 16 for ci in range(2)}

        def fetch(p, b):
            pltpu.async_copy(
                table_t.at[:, pl.ds(p * _PW, _PW)],
                sbuf.at[pl.ds(b * _D, _D), :],
                gsems[b],
            )

        def fetch_wait(p, b):
            pltpu.make_async_copy(
                table_t.at[:, pl.ds(p * _PW, _PW)],
                sbuf.at[pl.ds(b * _D, _D), :],
                gsems[b],
            ).wait()

        def transpose(b, nli):
            def body(li, carry):
                l0 = li * 16
                for ci in range(2):
                    row = rows[(b, ci)]
                    cb = cbase[ci]
                    for j in range(16):
                        col = diags[j] + l0
                        v = plsc.load_gather(sbuf, [row, col])
                        plsc.store_scatter(
                            lbuf,
                            [(col << 5) + cb + b * _PB],
                            v,
                        )
                return carry

            lax.fori_loop(0, nli, body, 0)

        def write(p, b):
            return (
                lbuf.at[pl.ds(b * _PB, _PB)],
                out_rm.at[pl.ds(p * _PB, _PB)],
                wsems[b],
            )

        fetch(wid, 0)
        fetch(wid + _NW, 1)

        def outer(u, carry):
            for b in range(2):
                t = u * 2 + b
                p = wid + t * _NW
                fetch_wait(p, b)

                @pl.when(u >= 1)
                def _():
                    pltpu.make_async_copy(*write(p - 2 * _NW, b)).wait()

                transpose(b, _PW // 16)
                pltpu.async_copy(*write(p, b))

                @pl.when(t + 2 < 244)
                def _():
                    fetch(p + 2 * _NW, b)

            return carry

        lax.fori_loop(0, 122, outer, 0)
        for b in range(2):
            pltpu.make_async_copy(*write(wid + (242 + b) * _NW, b)).wait()

        @pl.when(wid < 4)
        def _():
            p = 7808 + wid
            fetch(p, 0)
            fetch_wait(p, 0)
            transpose(0, _PW // 16)
            pltpu.async_copy(*write(p, 0))
            pltpu.make_async_copy(*write(p, 0)).wait()

        @pl.when(wid == 4)
        def _():
            pltpu.sync_copy(tail_in, sbuf.at[pl.ds(0, 16), pl.ds(0, _LN)])

            def body(r, carry):
                for seg in range(8):
                    v = plsc.load_gather(sbuf, [lax.broadcast(r, (16,)),
                                                iota + seg * 16])
                    plsc.store_scatter(lbuf, [iota + seg * 16 + r * _LN], v)
                return carry

            lax.fori_loop(0, 16, body, 0)
            pltpu.async_copy(
                lbuf.at[pl.ds(0, 64 * _D)],
                out_rm.at[pl.ds((_VOCAB - 64) * _D, 64 * _D)],
                wsems[0],
            )
            pltpu.make_async_copy(
                lbuf.at[pl.ds(0, 64 * _D)],
                out_rm.at[pl.ds((_VOCAB - 64) * _D, 64 * _D)],
                wsems[0],
            ).wait()

    return k


def _make_gather():
    mesh = plsc.VectorSubcoreMesh(core_axis_name="c", subcore_axis_name="s")

    @functools.partial(
        pl.kernel,
        mesh=mesh,
        out_type=jax.ShapeDtypeStruct((_H, _D // 8, _NB // _LN, 8, _LN),
                                      jnp.float32),
        scratch_types=[
            pltpu.VMEM((2 * _LN,), jnp.int32),
            pltpu.VMEM((2 * _LN, _D), jnp.float32),
            pltpu.VMEM((8, 8, _LN), jnp.float32),
            [pltpu.SemaphoreType.DMA] * 2,
            [pltpu.SemaphoreType.DMA] * 2,
        ],
        compiler_params=pltpu.CompilerParams(
            use_tc_tiling_on_sc=False, needs_layout_passes=False
        ),
    )
    def k(table_rm, idx_t, out5, idx_v, gbuf, tbuf, gsems, wsems):
        wid = lax.axis_index("s") * 2 + lax.axis_index("c")
        iota = _iota16()
        diags = [(iota + j) & 15 for j in range(16)]
        t_c8 = iota % 8
        t_ct = {ci: iota // 8 + ci * 2 for ci in range(2)}
        cvec = {ci: iota + ci * 16 for ci in range(2)}

        def stage(h, b):
            pltpu.sync_copy(
                idx_t.at[h, pl.ds(wid * _LN, _LN)],
                idx_v.at[pl.ds(b * _LN, _LN)],
            )
            pltpu.async_copy(
                table_rm.at[idx_v.at[pl.ds(b * _LN, _LN)]],
                gbuf.at[pl.ds(b * _LN, _LN), :],
                gsems[b],
            )

        def gather_wait(b):
            pltpu.make_async_copy(
                table_rm.at[idx_v.at[pl.ds(b * _LN, _LN)]],
                gbuf.at[pl.ds(b * _LN, _LN), :],
                gsems[b],
            ).wait()

        def wcopy(h, b):
            return (
                tbuf.at[pl.ds(b * 4, 4)],
                out5.at[h, :, wid],
                wsems[b],
            )

        stage(0, 0)

        def outer(u, carry):
            for b in range(2):
                h = u * 2 + b
                nb = 1 - b
                gather_wait(b)

                @pl.when(h + 1 < _H)
                def _():
                    stage(h + 1, nb)

                @pl.when(h >= 2)
                def _():
                    pltpu.make_async_copy(*wcopy(h - 2, b)).wait()

                def trow(lp, carry2):
                    l0 = lp * 16
                    for ci in range(2):
                        for j in range(16):
                            lvec = diags[j] + l0
                            v = plsc.load_gather(
                                gbuf, [lvec + b * _LN, cvec[ci]]
                            )
                            plsc.store_scatter(
                                tbuf,
                                [t_ct[ci] + b * 4, t_c8, lvec],
                                v,
                            )
                    return carry2

                lax.fori_loop(0, _LN // 16, trow, 0)
                pltpu.async_copy(*wcopy(h, b))
            return carry

        lax.fori_loop(0, _H // 2, outer, 0)
        for h in (_H - 2, _H - 1):
            pltpu.make_async_copy(*wcopy(h, h % 2)).wait()

    return k


_repack = _make_repack()
_gather = _make_gather()


def kernel(inputs, embeddings):
    tail128 = embeddings[_VOCAB - 64:, :].reshape(16, _LN)
    table_rm = _repack(embeddings.T, tail128)
    out5 = _gather(
        table_rm.reshape(_VOCAB, _D), inputs.T.astype(jnp.int32)
    )
    return out5.transpose(2, 4, 0, 1, 3).reshape(_NB, _H, _D)
